# baseline (speedup 1.0000x reference)
import functools

import jax
import jax.numpy as jnp
from jax import lax
from jax.experimental import pallas as pl
from jax.experimental.pallas import tpu as pltpu
from jax.experimental.pallas import tpu_sc as plsc

B = 16384
D = 32
H = 128
O = 5
BB = 2048


@functools.cache
def _gather_fn():
    info = plsc.get_sparse_core_info()
    NC, NS = info.num_cores, info.num_subcores
    NW = NC * NS
    b_per_w = B // NW
    mesh = plsc.VectorSubcoreMesh(core_axis_name="c", subcore_axis_name="s")

    @functools.partial(
        pl.kernel,
        mesh=mesh,
        out_type=jax.ShapeDtypeStruct((2 * D, B), jnp.float32),
        scratch_types=[
            pltpu.VMEM((D, b_per_w), jnp.int32),
            pltpu.VMEM((D, b_per_w), jnp.int32),
            pltpu.VMEM((D, b_per_w), jnp.float32),
            pltpu.VMEM((D, b_per_w), jnp.float32),
            pltpu.SemaphoreType.DMA,
        ],
        compiler_params=pltpu.CompilerParams(use_tc_tiling_on_sc=True),
    )
    def gather_k(ut1d, mt1d, uflat, mflat, emt_out,
                 uidx, midx, ucols, mcols, sem):
        wid = lax.axis_index("s") * NC + lax.axis_index("c")
        base = wid * b_per_w
        pltpu.sync_copy(uflat.at[:, pl.ds(base, b_per_w)], uidx)
        pltpu.sync_copy(mflat.at[:, pl.ds(base, b_per_w)], midx)

        for d in range(D):
            for q in range(b_per_w // 128):
                c = pl.ds(q * 128, 128)
                pltpu.async_copy(ut1d.at[uidx.at[d, c]], ucols.at[d, c], sem)
                pltpu.async_copy(mt1d.at[midx.at[d, c]], mcols.at[d, c], sem)
        dummy = emt_out.at[pl.ds(0, D), pl.ds(0, b_per_w)]
        pltpu.make_async_copy(dummy, ucols, sem).wait()
        pltpu.make_async_copy(dummy, mcols, sem).wait()
        pltpu.sync_copy(ucols, emt_out.at[pl.ds(0, D), pl.ds(base, b_per_w)])
        pltpu.sync_copy(mcols, emt_out.at[pl.ds(D, D), pl.ds(base, b_per_w)])

    return gather_k


def _mlp_body(emt, w1, b1, w2, b2, wout, bout, out):
    h1 = jnp.maximum(
        jnp.dot(w1[...], emt[...], preferred_element_type=jnp.float32)
        + b1[...], 0.0)
    h2 = jnp.maximum(
        jnp.dot(w2[...], h1, preferred_element_type=jnp.float32) + b2[...], 0.0)
    out[...] = jnp.dot(wout[...], h2, preferred_element_type=jnp.float32) + bout[...]


def kernel(User_ID, Movie_ID, Rating, user_table, movie_table,
           W1, b1, W2, b2, Wout, bout):
    nu = user_table.shape[0]
    nm = movie_table.shape[0]
    ut1d = user_table.T.reshape(-1)
    mt1d = movie_table.T.reshape(-1)
    dims = jnp.arange(D, dtype=jnp.int32).reshape(D, 1)
    uflat = dims * nu + User_ID.reshape(1, B)
    mflat = dims * nm + Movie_ID.reshape(1, B)
    emt = _gather_fn()(ut1d, mt1d, uflat, mflat)

    outt = pl.pallas_call(
        _mlp_body,
        grid=(B // BB,),
        in_specs=[
            pl.BlockSpec((2 * D, BB), lambda i: (0, i)),
            pl.BlockSpec((H, 2 * D), lambda i: (0, 0)),
            pl.BlockSpec((H, 1), lambda i: (0, 0)),
            pl.BlockSpec((H, H), lambda i: (0, 0)),
            pl.BlockSpec((H, 1), lambda i: (0, 0)),
            pl.BlockSpec((O, H), lambda i: (0, 0)),
            pl.BlockSpec((O, 1), lambda i: (0, 0)),
        ],
        out_specs=pl.BlockSpec((O, BB), lambda i: (0, i)),
        out_shape=jax.ShapeDtypeStruct((O, B), jnp.float32),
    )(emt, W1, b1.reshape(H, 1), W2, b2.reshape(H, 1),
      Wout, bout.reshape(O, 1))
    return outt.T


# speedup vs baseline: 4.5742x; 4.5742x over previous
import functools

import jax
import jax.numpy as jnp
from jax import lax
from jax.experimental import pallas as pl
from jax.experimental.pallas import tpu as pltpu
from jax.experimental.pallas import tpu_sc as plsc

B = 16384
D = 32
H = 128
O = 5
BB = 2048


@functools.cache
def _gather_fn():
    info = plsc.get_sparse_core_info()
    NC, NS = info.num_cores, info.num_subcores
    NW = NC * NS
    b_per_w = B // NW
    mesh = plsc.VectorSubcoreMesh(core_axis_name="c", subcore_axis_name="s")

    @functools.partial(
        pl.kernel,
        mesh=mesh,
        out_type=(
            jax.ShapeDtypeStruct((B, D), jnp.float32),
            jax.ShapeDtypeStruct((B, D), jnp.float32),
        ),
        scratch_types=[
            pltpu.VMEM((b_per_w,), jnp.int32),
            pltpu.VMEM((b_per_w,), jnp.int32),
            pltpu.VMEM((b_per_w, D), jnp.float32),
            pltpu.VMEM((b_per_w, D), jnp.float32),
            pltpu.SemaphoreType.DMA,
        ],
        compiler_params=pltpu.CompilerParams(use_tc_tiling_on_sc=False),
    )
    def gather_k(utab, mtab, uid, mid, ue_out, me_out,
                 uidx, midx, urows, mrows, sem):
        wid = lax.axis_index("s") * NC + lax.axis_index("c")
        base = wid * b_per_w
        pltpu.sync_copy(uid.at[pl.ds(base, b_per_w)], uidx)
        pltpu.sync_copy(mid.at[pl.ds(base, b_per_w)], midx)
        cu = pltpu.async_copy(utab.at[uidx], urows, sem)
        cm = pltpu.async_copy(mtab.at[midx], mrows, sem)
        cu.wait()
        cm.wait()
        pltpu.sync_copy(urows, ue_out.at[pl.ds(base, b_per_w)])
        pltpu.sync_copy(mrows, me_out.at[pl.ds(base, b_per_w)])

    return gather_k


def _mlp_body(ue, me, w1, b1, w2, b2, wout, bout, out):
    dn = (((1,), (1,)), ((), ()))
    h1 = lax.dot_general(ue[...], w1[:, :D], dn,
                         preferred_element_type=jnp.float32)
    h1 = h1 + lax.dot_general(me[...], w1[:, D:], dn,
                              preferred_element_type=jnp.float32)
    h1 = jnp.maximum(h1 + b1[...], 0.0)
    h2 = jnp.maximum(
        lax.dot_general(h1, w2[...], dn, preferred_element_type=jnp.float32)
        + b2[...], 0.0)
    out[...] = lax.dot_general(
        h2, wout[...], dn, preferred_element_type=jnp.float32) + bout[...]


def kernel(User_ID, Movie_ID, Rating, user_table, movie_table,
           W1, b1, W2, b2, Wout, bout):
    uid = User_ID.astype(jnp.int32)
    mid = Movie_ID.astype(jnp.int32)
    ue, me = _gather_fn()(user_table, movie_table, uid, mid)

    out = pl.pallas_call(
        _mlp_body,
        grid=(B // BB,),
        in_specs=[
            pl.BlockSpec((BB, D), lambda i: (i, 0)),
            pl.BlockSpec((BB, D), lambda i: (i, 0)),
            pl.BlockSpec((H, 2 * D), lambda i: (0, 0)),
            pl.BlockSpec((1, H), lambda i: (0, 0)),
            pl.BlockSpec((H, H), lambda i: (0, 0)),
            pl.BlockSpec((1, H), lambda i: (0, 0)),
            pl.BlockSpec((O, H), lambda i: (0, 0)),
            pl.BlockSpec((1, O), lambda i: (0, 0)),
        ],
        out_specs=pl.BlockSpec((BB, O), lambda i: (i, 0)),
        out_shape=jax.ShapeDtypeStruct((B, O), jnp.float32),
    )(ue, me, W1, b1.reshape(1, H), W2, b2.reshape(1, H),
      Wout, bout.reshape(1, O))
    return out
